# baseline (speedup 1.0000x reference)
import functools

import jax
import jax.numpy as jnp
from jax.experimental import pallas as pl
from jax.experimental.pallas import tpu as pltpu
from jax.experimental.pallas import tpu_sc as plsc

E = 16
TOPK = 2
NGROUP = 4
GSIZE = E // NGROUP
H = 2048
I = 1024
RSF = 2.5
NEG = -1e30
IB = 512
NC = 2
NS = 16
NW = NC * NS


def _silu(x):
    return x * jax.nn.sigmoid(x)


def _bf16_dot(a, b, dn):
    return jax.lax.dot_general(a, b, dn,
                               precision=jax.lax.Precision.DEFAULT,
                               preferred_element_type=jnp.float32)


_DN_T = (((1,), (1,)), ((), ()))


def _scores_kernel(x_ref, gw_ref, bias_ref, scores_ref, sfc_ref):
    logits = jax.lax.dot_general(x_ref[...], gw_ref[...], _DN_T,
                                 preferred_element_type=jnp.float32)
    scores = jax.nn.sigmoid(logits)
    scores_ref[...] = scores
    sfc_ref[...] = scores + bias_ref[...]


def _vmax(x):
    return plsc.cummax(jax.lax.rev(plsc.cummax(x), (0,)))


def _ffs(mask):
    return plsc.all_reduce_ffs(mask)


def _route_token(s, sc, iota, gid):
    gsum = jnp.where(gid < 0, s, 0.0)
    for g in range(NGROUP):
        m = gid == g
        sg = jnp.where(m, s, NEG)
        m1 = _vmax(sg)
        i1 = _ffs(sg == m1)
        m2 = _vmax(jnp.where(iota == i1, NEG, sg))
        gsum = jnp.where(m, m1 + m2, gsum)
    g1 = jax.lax.shift_right_logical(_ffs(gsum == _vmax(gsum)), 2)
    rem = jnp.where(gid == g1, NEG, gsum)
    g2 = jax.lax.shift_right_logical(_ffs(rem == _vmax(rem)), 2)
    sel = (gid == g1) | (gid == g2)
    ms = jnp.where(sel, s, 0.0)
    e1 = _ffs(ms == _vmax(ms))
    ms2 = jnp.where(iota == e1, NEG, ms)
    e2 = _ffs(ms2 == _vmax(ms2))
    w1 = _vmax(jnp.where(iota == e1, sc, 0.0))
    w2 = _vmax(jnp.where(iota == e2, sc, 0.0))
    d = w1 + w2 + 1e-20
    return (jnp.where(iota == e1, w1 / d * RSF, 0.0)
            + jnp.where(iota == e2, w2 / d * RSF, 0.0))


def _route_on_sc(scores, sfc):
    T = scores.shape[0]
    tpw = T // NW
    mesh = plsc.VectorSubcoreMesh(core_axis_name="c", subcore_axis_name="s")

    @functools.partial(
        pl.kernel, mesh=mesh,
        out_type=jax.ShapeDtypeStruct((T, E), jnp.float32),
        scratch_types=[
            pltpu.VMEM((tpw, E), jnp.float32),
            pltpu.VMEM((tpw, E), jnp.float32),
            pltpu.VMEM((tpw, E), jnp.float32),
        ],
        compiler_params=pltpu.CompilerParams(needs_layout_passes=False),
    )
    def k(scores_hbm, sfc_hbm, out_hbm, sc_v, sfc_v, out_v):
        wid = jax.lax.axis_index("s") * NC + jax.lax.axis_index("c")
        base = wid * tpw
        pltpu.sync_copy(scores_hbm.at[pl.ds(base, tpw)], sc_v)
        pltpu.sync_copy(sfc_hbm.at[pl.ds(base, tpw)], sfc_v)
        iota = jax.lax.iota(jnp.int32, E)
        gid = jax.lax.shift_right_logical(iota, 2)
        for i in range(tpw):
            out_v[i] = _route_token(sfc_v[i], sc_v[i], iota, gid)
        pltpu.sync_copy(out_v, out_hbm.at[pl.ds(base, tpw)])

    return k(scores, sfc)


def _moe_kernel(x_ref, we_ref, wg_ref, wu_ref, wd_ref,
                swg_ref, swu_ref, swd_ref, out_ref):
    e = pl.program_id(0)
    ki = pl.program_id(1)
    x = x_ref[...]

    @pl.when((e == 0) & (ki == 0))
    def _first():
        sg = _bf16_dot(x, swg_ref[...], _DN_T)
        su = _bf16_dot(x, swu_ref[...], _DN_T)
        out_ref[...] = _bf16_dot(_silu(sg) * su, swd_ref[...], _DN_T)

    g = _bf16_dot(x, wg_ref[0], _DN_T)
    u = _bf16_dot(x, wu_ref[0], _DN_T)
    h = _silu(g) * u
    y = _bf16_dot(h, wd_ref[0], _DN_T)
    iota = jax.lax.broadcasted_iota(jnp.int32, we_ref.shape, 1)
    w_col = jnp.sum(jnp.where(iota == e, we_ref[...], 0.0), axis=1,
                    keepdims=True)
    out_ref[...] += w_col * y


def kernel(hidden_states, gate_weight, correction_bias, w_gate, w_up, w_down,
           sw_gate, sw_up, sw_down, num_global_tokens, max_num_tokens_per_gpu):
    T = hidden_states.shape[0]
    bias2d = correction_bias.reshape(1, E)

    scores, sfc = pl.pallas_call(
        _scores_kernel,
        out_shape=[jax.ShapeDtypeStruct((T, E), jnp.float32),
                   jax.ShapeDtypeStruct((T, E), jnp.float32)],
    )(hidden_states, gate_weight, bias2d)

    we = _route_on_sc(scores, sfc)

    n_ki = I // IB
    grid = (E, n_ki)
    return pl.pallas_call(
        _moe_kernel,
        grid=grid,
        in_specs=[
            pl.BlockSpec((T, H), lambda e, k: (0, 0)),
            pl.BlockSpec((T, E), lambda e, k: (0, 0)),
            pl.BlockSpec((1, IB, H), lambda e, k: (e, k, 0)),
            pl.BlockSpec((1, IB, H), lambda e, k: (e, k, 0)),
            pl.BlockSpec((1, H, IB), lambda e, k: (e, 0, k)),
            pl.BlockSpec((I, H), lambda e, k: (0, 0)),
            pl.BlockSpec((I, H), lambda e, k: (0, 0)),
            pl.BlockSpec((H, I), lambda e, k: (0, 0)),
        ],
        out_specs=pl.BlockSpec((T, H), lambda e, k: (0, 0)),
        out_shape=jax.ShapeDtypeStruct((T, H), jnp.float32),
        compiler_params=pltpu.CompilerParams(
            dimension_semantics=("arbitrary", "arbitrary")),
    )(hidden_states, we, w_gate, w_up, w_down, sw_gate, sw_up, sw_down)


# speedup vs baseline: 1.0333x; 1.0333x over previous
import functools

import jax
import jax.numpy as jnp
from jax.experimental import pallas as pl
from jax.experimental.pallas import tpu as pltpu
from jax.experimental.pallas import tpu_sc as plsc

E = 16
TOPK = 2
NGROUP = 4
GSIZE = E // NGROUP
H = 2048
I = 1024
RSF = 2.5
NEG = -1e30
IB = 512
NC = 2
NS = 16
NW = NC * NS


def _silu(x):
    return x * jax.nn.sigmoid(x)


def _bf16_dot(a, b, dn):
    return jax.lax.dot_general(a, b, dn,
                               precision=jax.lax.Precision.DEFAULT,
                               preferred_element_type=jnp.float32)


_DN_T = (((1,), (1,)), ((), ()))


def _scores_kernel(x_ref, gw_ref, bias_ref, scores_ref, sfc_ref):
    logits = jax.lax.dot_general(x_ref[...], gw_ref[...], _DN_T,
                                 preferred_element_type=jnp.float32)
    scores = jax.nn.sigmoid(logits)
    scores_ref[...] = scores
    sfc_ref[...] = scores + bias_ref[...]


def _vmax(x):
    return plsc.cummax(jax.lax.rev(plsc.cummax(x), (0,)))


def _ffs(mask):
    return plsc.all_reduce_ffs(mask)


def _route_token(s, sc, iota, gid):
    gsum = jnp.where(gid < 0, s, 0.0)
    for g in range(NGROUP):
        m = gid == g
        sg = jnp.where(m, s, NEG)
        m1 = _vmax(sg)
        i1 = _ffs(sg == m1)
        m2 = _vmax(jnp.where(iota == i1, NEG, sg))
        gsum = jnp.where(m, m1 + m2, gsum)
    g1 = jax.lax.shift_right_logical(_ffs(gsum == _vmax(gsum)), 2)
    rem = jnp.where(gid == g1, NEG, gsum)
    g2 = jax.lax.shift_right_logical(_ffs(rem == _vmax(rem)), 2)
    sel = (gid == g1) | (gid == g2)
    ms = jnp.where(sel, s, 0.0)
    e1 = _ffs(ms == _vmax(ms))
    ms2 = jnp.where(iota == e1, NEG, ms)
    e2 = _ffs(ms2 == _vmax(ms2))
    w1 = _vmax(jnp.where(iota == e1, sc, 0.0))
    w2 = _vmax(jnp.where(iota == e2, sc, 0.0))
    d = w1 + w2 + 1e-20
    return (jnp.where(iota == e1, w1 / d * RSF, 0.0)
            + jnp.where(iota == e2, w2 / d * RSF, 0.0))


def _route_on_sc(scores, sfc):
    T = scores.shape[0]
    tpw = T // NW
    mesh = plsc.VectorSubcoreMesh(core_axis_name="c", subcore_axis_name="s")

    @functools.partial(
        pl.kernel, mesh=mesh,
        out_type=jax.ShapeDtypeStruct((T, E), jnp.float32),
        scratch_types=[
            pltpu.VMEM((tpw, E), jnp.float32),
            pltpu.VMEM((tpw, E), jnp.float32),
            pltpu.VMEM((tpw, E), jnp.float32),
        ],
        compiler_params=pltpu.CompilerParams(needs_layout_passes=False),
    )
    def k(scores_hbm, sfc_hbm, out_hbm, sc_v, sfc_v, out_v):
        wid = jax.lax.axis_index("s") * NC + jax.lax.axis_index("c")
        base = wid * tpw
        pltpu.sync_copy(scores_hbm.at[pl.ds(base, tpw)], sc_v)
        pltpu.sync_copy(sfc_hbm.at[pl.ds(base, tpw)], sfc_v)
        iota = jax.lax.iota(jnp.int32, E)
        gid = jax.lax.shift_right_logical(iota, 2)
        for i in range(tpw):
            out_v[i] = _route_token(sfc_v[i], sc_v[i], iota, gid)
        pltpu.sync_copy(out_v, out_hbm.at[pl.ds(base, tpw)])

    return k(scores, sfc)


SB = 256


def _shared_kernel(x_ref, swg_ref, swu_ref, swd_ref, out_ref):
    ki = pl.program_id(0)
    x = x_ref[...]
    sg = _bf16_dot(x, swg_ref[...], _DN_T)
    su = _bf16_dot(x, swu_ref[...], _DN_T)
    y = _bf16_dot(_silu(sg) * su, swd_ref[...], _DN_T)

    @pl.when(ki == 0)
    def _init():
        out_ref[...] = y

    @pl.when(ki > 0)
    def _acc():
        out_ref[...] += y


def _moe_kernel(x_ref, we_ref, shared_ref, wg_ref, wu_ref, wd_ref, out_ref):
    e = pl.program_id(0)
    ki = pl.program_id(1)
    x = x_ref[...]

    @pl.when((e == 0) & (ki == 0))
    def _first():
        out_ref[...] = shared_ref[...]

    g = _bf16_dot(x, wg_ref[0], _DN_T)
    u = _bf16_dot(x, wu_ref[0], _DN_T)
    h = _silu(g) * u
    y = _bf16_dot(h, wd_ref[0], _DN_T)
    iota = jax.lax.broadcasted_iota(jnp.int32, we_ref.shape, 1)
    w_col = jnp.sum(jnp.where(iota == e, we_ref[...], 0.0), axis=1,
                    keepdims=True)
    out_ref[...] += w_col * y


def kernel(hidden_states, gate_weight, correction_bias, w_gate, w_up, w_down,
           sw_gate, sw_up, sw_down, num_global_tokens, max_num_tokens_per_gpu):
    T = hidden_states.shape[0]
    bias2d = correction_bias.reshape(1, E)

    scores, sfc = pl.pallas_call(
        _scores_kernel,
        out_shape=[jax.ShapeDtypeStruct((T, E), jnp.float32),
                   jax.ShapeDtypeStruct((T, E), jnp.float32)],
    )(hidden_states, gate_weight, bias2d)

    we = _route_on_sc(scores, sfc)

    shared = pl.pallas_call(
        _shared_kernel,
        grid=(I // SB,),
        in_specs=[
            pl.BlockSpec((T, H), lambda k: (0, 0)),
            pl.BlockSpec((SB, H), lambda k: (k, 0)),
            pl.BlockSpec((SB, H), lambda k: (k, 0)),
            pl.BlockSpec((H, SB), lambda k: (0, k)),
        ],
        out_specs=pl.BlockSpec((T, H), lambda k: (0, 0)),
        out_shape=jax.ShapeDtypeStruct((T, H), jnp.float32),
        compiler_params=pltpu.CompilerParams(
            dimension_semantics=("arbitrary",)),
    )(hidden_states, sw_gate, sw_up, sw_down)

    n_ki = I // IB
    grid = (E, n_ki)
    return pl.pallas_call(
        _moe_kernel,
        grid=grid,
        in_specs=[
            pl.BlockSpec((T, H), lambda e, k: (0, 0)),
            pl.BlockSpec((T, E), lambda e, k: (0, 0)),
            pl.BlockSpec((T, H), lambda e, k: (0, 0)),
            pl.BlockSpec((1, IB, H), lambda e, k: (e, k, 0)),
            pl.BlockSpec((1, IB, H), lambda e, k: (e, k, 0)),
            pl.BlockSpec((1, H, IB), lambda e, k: (e, 0, k)),
        ],
        out_specs=pl.BlockSpec((T, H), lambda e, k: (0, 0)),
        out_shape=jax.ShapeDtypeStruct((T, H), jnp.float32),
        compiler_params=pltpu.CompilerParams(
            dimension_semantics=("arbitrary", "arbitrary")),
    )(hidden_states, we, shared, w_gate, w_up, w_down)


# speedup vs baseline: 1.0348x; 1.0015x over previous
import functools

import jax
import jax.numpy as jnp
from jax.experimental import pallas as pl
from jax.experimental.pallas import tpu as pltpu
from jax.experimental.pallas import tpu_sc as plsc

E = 16
TOPK = 2
NGROUP = 4
GSIZE = E // NGROUP
H = 2048
I = 1024
RSF = 2.5
NEG = -1e30
IB = 512
NC = 2
NS = 16
NW = NC * NS


def _silu(x):
    return x * jax.nn.sigmoid(x)


def _bf16_dot(a, b, dn):
    return jax.lax.dot_general(a, b, dn,
                               precision=jax.lax.Precision.DEFAULT,
                               preferred_element_type=jnp.float32)


_DN_T = (((1,), (1,)), ((), ()))


def _scores_kernel(x_ref, gw_ref, bias_ref, scores_ref, sfc_ref):
    logits = jax.lax.dot_general(x_ref[...], gw_ref[...], _DN_T,
                                 preferred_element_type=jnp.float32)
    scores = jax.nn.sigmoid(logits)
    scores_ref[...] = scores
    sfc_ref[...] = scores + bias_ref[...]


def _vmax(x):
    return plsc.cummax(jax.lax.rev(plsc.cummax(x), (0,)))


def _ffs(mask):
    return plsc.all_reduce_ffs(mask)


def _route_token(s, sc, iota, gid):
    gsum = jnp.where(gid < 0, s, 0.0)
    for g in range(NGROUP):
        m = gid == g
        sg = jnp.where(m, s, NEG)
        m1 = _vmax(sg)
        i1 = _ffs(sg == m1)
        m2 = _vmax(jnp.where(iota == i1, NEG, sg))
        gsum = jnp.where(m, m1 + m2, gsum)
    g1 = jax.lax.shift_right_logical(_ffs(gsum == _vmax(gsum)), 2)
    rem = jnp.where(gid == g1, NEG, gsum)
    g2 = jax.lax.shift_right_logical(_ffs(rem == _vmax(rem)), 2)
    sel = (gid == g1) | (gid == g2)
    ms = jnp.where(sel, s, 0.0)
    e1 = _ffs(ms == _vmax(ms))
    ms2 = jnp.where(iota == e1, NEG, ms)
    e2 = _ffs(ms2 == _vmax(ms2))
    w1 = _vmax(jnp.where(iota == e1, sc, 0.0))
    w2 = _vmax(jnp.where(iota == e2, sc, 0.0))
    d = w1 + w2 + 1e-20
    return (jnp.where(iota == e1, w1 / d * RSF, 0.0)
            + jnp.where(iota == e2, w2 / d * RSF, 0.0))


def _route_on_sc(scores, sfc):
    T = scores.shape[0]
    tpw = T // NW
    mesh = plsc.VectorSubcoreMesh(core_axis_name="c", subcore_axis_name="s")

    @functools.partial(
        pl.kernel, mesh=mesh,
        out_type=jax.ShapeDtypeStruct((T, E), jnp.float32),
        scratch_types=[
            pltpu.VMEM((tpw, E), jnp.float32),
            pltpu.VMEM((tpw, E), jnp.float32),
            pltpu.VMEM((tpw, E), jnp.float32),
        ],
        compiler_params=pltpu.CompilerParams(needs_layout_passes=False),
    )
    def k(scores_hbm, sfc_hbm, out_hbm, sc_v, sfc_v, out_v):
        wid = jax.lax.axis_index("s") * NC + jax.lax.axis_index("c")
        base = wid * tpw
        pltpu.sync_copy(scores_hbm.at[pl.ds(base, tpw)], sc_v)
        pltpu.sync_copy(sfc_hbm.at[pl.ds(base, tpw)], sfc_v)
        iota = jax.lax.iota(jnp.int32, E)
        gid = jax.lax.shift_right_logical(iota, 2)

        def body(i, _):
            out_v[i] = _route_token(sfc_v[i], sc_v[i], iota, gid)
            return 0

        jax.lax.fori_loop(0, tpw, body, 0)
        pltpu.sync_copy(out_v, out_hbm.at[pl.ds(base, tpw)])

    return k(scores, sfc)


SB = 256


def _shared_kernel(x_ref, swg_ref, swu_ref, swd_ref, out_ref):
    ki = pl.program_id(0)
    x = x_ref[...]
    sg = _bf16_dot(x, swg_ref[...], _DN_T)
    su = _bf16_dot(x, swu_ref[...], _DN_T)
    y = _bf16_dot(_silu(sg) * su, swd_ref[...], _DN_T)

    @pl.when(ki == 0)
    def _init():
        out_ref[...] = y

    @pl.when(ki > 0)
    def _acc():
        out_ref[...] += y


def _moe_kernel(x_ref, we_ref, shared_ref, wg_ref, wu_ref, wd_ref, out_ref):
    e = pl.program_id(0)
    ki = pl.program_id(1)
    x = x_ref[...]

    @pl.when((e == 0) & (ki == 0))
    def _first():
        out_ref[...] = shared_ref[...]

    g = _bf16_dot(x, wg_ref[0], _DN_T)
    u = _bf16_dot(x, wu_ref[0], _DN_T)
    h = _silu(g) * u
    y = _bf16_dot(h, wd_ref[0], _DN_T)
    iota = jax.lax.broadcasted_iota(jnp.int32, we_ref.shape, 1)
    w_col = jnp.sum(jnp.where(iota == e, we_ref[...], 0.0), axis=1,
                    keepdims=True)
    out_ref[...] += w_col * y


def kernel(hidden_states, gate_weight, correction_bias, w_gate, w_up, w_down,
           sw_gate, sw_up, sw_down, num_global_tokens, max_num_tokens_per_gpu):
    T = hidden_states.shape[0]
    bias2d = correction_bias.reshape(1, E)

    scores, sfc = pl.pallas_call(
        _scores_kernel,
        out_shape=[jax.ShapeDtypeStruct((T, E), jnp.float32),
                   jax.ShapeDtypeStruct((T, E), jnp.float32)],
    )(hidden_states, gate_weight, bias2d)

    we = _route_on_sc(scores, sfc)

    shared = pl.pallas_call(
        _shared_kernel,
        grid=(I // SB,),
        in_specs=[
            pl.BlockSpec((T, H), lambda k: (0, 0)),
            pl.BlockSpec((SB, H), lambda k: (k, 0)),
            pl.BlockSpec((SB, H), lambda k: (k, 0)),
            pl.BlockSpec((H, SB), lambda k: (0, k)),
        ],
        out_specs=pl.BlockSpec((T, H), lambda k: (0, 0)),
        out_shape=jax.ShapeDtypeStruct((T, H), jnp.float32),
        compiler_params=pltpu.CompilerParams(
            dimension_semantics=("arbitrary",)),
    )(hidden_states, sw_gate, sw_up, sw_down)

    n_ki = I // IB
    grid = (E, n_ki)
    return pl.pallas_call(
        _moe_kernel,
        grid=grid,
        in_specs=[
            pl.BlockSpec((T, H), lambda e, k: (0, 0)),
            pl.BlockSpec((T, E), lambda e, k: (0, 0)),
            pl.BlockSpec((T, H), lambda e, k: (0, 0)),
            pl.BlockSpec((1, IB, H), lambda e, k: (e, k, 0)),
            pl.BlockSpec((1, IB, H), lambda e, k: (e, k, 0)),
            pl.BlockSpec((1, H, IB), lambda e, k: (e, 0, k)),
        ],
        out_specs=pl.BlockSpec((T, H), lambda e, k: (0, 0)),
        out_shape=jax.ShapeDtypeStruct((T, H), jnp.float32),
        compiler_params=pltpu.CompilerParams(
            dimension_semantics=("arbitrary", "arbitrary")),
    )(hidden_states, we, shared, w_gate, w_up, w_down)
